# baseline (speedup 1.0000x reference)
import functools

import jax
import jax.numpy as jnp
from jax.experimental import pallas as pl
from jax.experimental.pallas import tpu as pltpu

_N = 4
_P = 4096
_BP = 1024
_R = _P // _BP


def _chamfer_kernel(p1_ref, p2_ref, out_ref, colmin_ref, acc_ref, baug_ref):
    n = pl.program_id(0)
    r = pl.program_id(1)

    a = p1_ref[0]

    a2 = jnp.sum(a * a, axis=1, keepdims=True)
    a2_hi = a2.astype(jnp.bfloat16).astype(jnp.float32)
    a2_lo = a2 - a2_hi

    a_hi = a.astype(jnp.bfloat16).astype(jnp.float32)
    a_lo = a - a_hi
    la = jax.lax.broadcasted_iota(jnp.int32, a.shape, 1)

    m2h = -2.0 * a_hi
    a_aug = jnp.where(la < 3, m2h,
            jnp.where(la < 6, jnp.roll(-2.0 * a_lo, 3, axis=1),
            jnp.where(la < 9, jnp.roll(m2h, 6, axis=1),
            jnp.where(la == 9, a2_hi,
            jnp.where(la == 10, a2_lo,
            ((la == 11) | (la == 12)).astype(jnp.float32))))))

    @pl.when(r == 0)
    def _build_baug():
        b = p2_ref[0]
        b2 = jnp.sum(b * b, axis=1, keepdims=True)
        b2_hi = b2.astype(jnp.bfloat16).astype(jnp.float32)
        b2_lo = b2 - b2_hi
        b_hi = b.astype(jnp.bfloat16).astype(jnp.float32)
        b_lo = b - b_hi
        lb = jax.lax.broadcasted_iota(jnp.int32, b.shape, 1)
        baug_ref[...] = jnp.where(lb < 3, b_hi,
                        jnp.where(lb < 6, jnp.roll(b_hi, 3, axis=1),
                        jnp.where(lb < 9, jnp.roll(b_lo, 6, axis=1),
                        jnp.where(lb == 11, b2_hi,
                        jnp.where(lb == 12, b2_lo,
                        ((lb == 9) | (lb == 10)).astype(jnp.float32))))))

    d = jax.lax.dot_general(
        a_aug, baug_ref[...], (((1,), (1,)), ((), ())),
        preferred_element_type=jnp.float32,
    )

    row_min = jnp.min(d, axis=1)
    col_min = jnp.min(d, axis=0, keepdims=True)

    @pl.when(jnp.logical_and(n == 0, r == 0))
    def _init_acc():
        acc_ref[...] = jnp.zeros((1, 1), jnp.float32)

    @pl.when(r == 0)
    def _init_colmin():
        colmin_ref[...] = col_min

    @pl.when(r > 0)
    def _merge_colmin():
        colmin_ref[...] = jnp.minimum(colmin_ref[...], col_min)

    acc_ref[...] += jnp.sum(row_min)[None, None]

    @pl.when(r == _R - 1)
    def _fold_colmin():
        acc_ref[...] += jnp.sum(colmin_ref[...])[None, None]

    @pl.when(jnp.logical_and(n == _N - 1, r == _R - 1))
    def _finalize():
        out_ref[...] = acc_ref[...] * (1.0 / (_N * _P))


@jax.jit
def kernel(p1, p2):
    p1p = jnp.pad(p1, ((0, 0), (0, 0), (0, 13)))
    p2p = jnp.pad(p2, ((0, 0), (0, 0), (0, 13)))

    out = pl.pallas_call(
        _chamfer_kernel,
        grid=(_N, _R),
        in_specs=[
            pl.BlockSpec((1, _BP, 16), lambda n, r: (n, r, 0)),
            pl.BlockSpec((1, _P, 16), lambda n, r: (n, 0, 0)),
        ],
        out_specs=pl.BlockSpec((1, 1), lambda n, r: (0, 0)),
        out_shape=jax.ShapeDtypeStruct((1, 1), jnp.float32),
        scratch_shapes=[
            pltpu.VMEM((1, _P), jnp.float32),
            pltpu.VMEM((1, 1), jnp.float32),
            pltpu.VMEM((_P, 16), jnp.float32),
        ],
    )(p1p, p2p)
    return out[0, 0]
